# baseline (speedup 1.0000x reference)
import functools

import jax
import jax.numpy as jnp
from jax import lax
from jax.experimental import pallas as pl
from jax.experimental.pallas import tpu as pltpu
from jax.experimental.pallas import tpu_sc as plsc

B = 4096
T = 200
D = 64
N = B * T

NC = 2
NS = 16
NW = NC * NS
BBLK = B // NW

G = 50
NGRP = T // G
NBUF = 2


def _make_kernel():
    mesh = plsc.VectorSubcoreMesh(core_axis_name="c", subcore_axis_name="s")

    scratch = (
        [
            pltpu.VMEM((1000, D), jnp.float32),
            pltpu.VMEM((201, D), jnp.float32),
            pltpu.VMEM((T, D), jnp.float32),
            pltpu.VMEM((G, BBLK), jnp.int32),
            pltpu.VMEM((G, BBLK), jnp.int32),
        ]
        + [pltpu.VMEM((8, 8, 128), jnp.float32) for _ in range(NBUF)]
        + [pltpu.SemaphoreType.DMA for _ in range(NBUF)]
    )

    @functools.partial(
        pl.kernel,
        out_type=jax.ShapeDtypeStruct((T, 8, NW, 8, 128), jnp.float32),
        mesh=mesh,
        scratch_types=scratch,
        compiler_params=pltpu.CompilerParams(
            use_tc_tiling_on_sc=False, needs_layout_passes=False),
    )
    def embed_kernel(xtT_hbm, ytT_hbm, xtab_hbm, ytab_hbm, ptab_hbm, out_hbm,
                     xtab, ytab, ptab, tokx, toky, buf0, buf1, sem0, sem1):
        bufs = (buf0, buf1)
        sems = (sem0, sem1)
        wid = lax.axis_index("s") * NC + lax.axis_index("c")
        bs = wid * BBLK

        pltpu.sync_copy(xtab_hbm, xtab)
        pltpu.sync_copy(ytab_hbm, ytab)
        pltpu.sync_copy(ptab_hbm, ptab)

        def group(g, _):
            t0 = g * G
            pltpu.sync_copy(
                xtT_hbm.at[pl.ds(t0, G), pl.ds(bs, BBLK)], tokx)
            pltpu.sync_copy(
                ytT_hbm.at[pl.ds(t0, G), pl.ds(bs, BBLK)], toky)

            def pair(i, _):
                for p in range(NBUF):
                    tl = NBUF * i + p
                    t = t0 + tl
                    kglob = g * (G // NBUF) + i

                    @pl.when(kglob > 0)
                    def _(p=p):
                        for dt in range(8):
                            pltpu.make_async_copy(
                                out_hbm.at[0, dt, 0], bufs[p].at[dt], sems[p]
                            ).wait()

                    t_spl = jnp.full((16,), t, jnp.int32)
                    xt_vecs = [tokx[tl, pl.ds(j * 16, 16)] for j in range(8)]
                    yt_vecs = [toky[tl, pl.ds(j * 16, 16)] for j in range(8)]

                    def dloop(d, _, p=p, t_spl=t_spl, xt_vecs=xt_vecs,
                              yt_vecs=yt_vecs):
                        d_spl = jnp.full((16,), d, jnp.int32)
                        pv = plsc.load_gather(ptab, [t_spl, d_spl])
                        dt = d // 8
                        ds_ = d % 8
                        for j in range(8):
                            xv = plsc.load_gather(xtab, [xt_vecs[j], d_spl])
                            yv = plsc.load_gather(ytab, [yt_vecs[j], d_spl])
                            bufs[p][dt, ds_, pl.ds(j * 16, 16)] = xv + yv + pv
                        return 0

                    lax.fori_loop(0, D, dloop, 0)

                    for dt in range(8):
                        pltpu.async_copy(
                            bufs[p].at[dt], out_hbm.at[t, dt, wid], sems[p])
                return ()

            lax.fori_loop(0, G // NBUF, pair, ())
            return ()

        lax.fori_loop(0, NGRP, group, ())

        for p in range(NBUF):
            for dt in range(8):
                pltpu.make_async_copy(
                    out_hbm.at[0, dt, 0], bufs[p].at[dt], sems[p]
                ).wait()

    return embed_kernel


_kernel = _make_kernel()


@jax.jit
def kernel(x_tokens, y_tokens, x_table, y_table, pos_table):
    xtT = x_tokens.T.astype(jnp.int32)
    ytT = y_tokens.T.astype(jnp.int32)
    out5 = _kernel(xtT, ytT, x_table, y_table, pos_table)
    return jnp.transpose(out5, (2, 4, 0, 1, 3)).reshape(B, T, D)


# speedup vs baseline: 3.2330x; 3.2330x over previous
import functools

import jax
import jax.numpy as jnp
from jax import lax
from jax.experimental import pallas as pl
from jax.experimental.pallas import tpu as pltpu
from jax.experimental.pallas import tpu_sc as plsc

B = 4096
T = 200
D = 64
N = B * T

NC = 2
NS = 16
NW = NC * NS

ROWS_PER_W = B // NW
CHUNK = T
NCHUNK = ROWS_PER_W
NBUF = 4
K = 2
NSUPER = NCHUNK // NBUF


def _make_kernel():
    mesh = plsc.VectorSubcoreMesh(core_axis_name="c", subcore_axis_name="s")

    scratch = (
        [pltpu.VMEM((CHUNK,), jnp.int32) for _ in range(NBUF)]
        + [pltpu.VMEM((CHUNK,), jnp.int32) for _ in range(NBUF)]
        + [pltpu.VMEM((CHUNK, D), jnp.float32) for _ in range(NBUF)]
        + [pltpu.VMEM((CHUNK, D), jnp.float32) for _ in range(NBUF)]
        + [pltpu.VMEM((T, D), jnp.float32)]
        + [
            pltpu.VMEM_SHARED((1000, D), jnp.float32),
            pltpu.VMEM_SHARED((201, D), jnp.float32),
        ]
        + [pltpu.SemaphoreType.DMA for _ in range(3 * NBUF)]
    )

    @functools.partial(
        pl.kernel,
        out_type=jax.ShapeDtypeStruct((B, T, D), jnp.float32),
        mesh=mesh,
        scratch_types=scratch,
        compiler_params=pltpu.CompilerParams(use_tc_tiling_on_sc=False),
    )
    def embed_kernel(xt_hbm, yt_hbm, xtab_hbm, ytab_hbm, ptab_hbm, out_hbm,
                     *refs):
        xidx = refs[0:NBUF]
        yidx = refs[NBUF:2 * NBUF]
        bufX = refs[2 * NBUF:3 * NBUF]
        bufY = refs[3 * NBUF:4 * NBUF]
        tmpl = refs[4 * NBUF]
        xtab_sp, ytab_sp = refs[4 * NBUF + 1:4 * NBUF + 3]
        sems = refs[4 * NBUF + 3:]
        semA = sems[0:NBUF]
        semB = sems[NBUF:2 * NBUF]
        semD = sems[2 * NBUF:3 * NBUF]

        wid = lax.axis_index("s") * NC + lax.axis_index("c")
        base_tok = wid * (ROWS_PER_W * T)

        @pl.when(lax.axis_index("s") == 0)
        def _():
            pltpu.sync_copy(xtab_hbm, xtab_sp)
            pltpu.sync_copy(ytab_hbm, ytab_sp)

        pltpu.sync_copy(ptab_hbm, tmpl)
        plsc.subcore_barrier()

        def drain(sem, dst):
            pltpu.make_async_copy(out_hbm.at[0], dst, sem).wait()

        def drain_idx(sem, dst):
            pltpu.make_async_copy(xt_hbm.at[pl.ds(0, CHUNK)], dst, sem).wait()

        def combine_store(bj, rowj):
            drain(semB[bj], bufX[bj])
            drain(semB[bj], bufY[bj])

            def vrow(j, _):
                for r in range(4):
                    row = j * 4 + r
                    for k in range(D // 16):
                        sl = pl.ds(k * 16, 16)
                        bufX[bj][row, sl] = (
                            bufX[bj][row, sl]
                            + bufY[bj][row, sl]
                            + tmpl[row, sl]
                        )
                return ()

            lax.fori_loop(0, CHUNK // 4, vrow, ())

            pltpu.async_copy(bufX[bj], out_hbm.at[rowj], semD[bj])

        def superstep(g, _):
            for b in range(NBUF):
                ci = g * NBUF + b
                tok0 = base_tok + ci * CHUNK

                @pl.when(g > 0)
                def _(b=b):
                    drain(semD[b], bufX[b])
                    drain_idx(semA[b], xidx[b])
                    drain_idx(semA[b], yidx[b])

                @pl.when(g == 0)
                def _(b=b, tok0=tok0):
                    pltpu.sync_copy(xt_hbm.at[pl.ds(tok0, CHUNK)], xidx[b])
                    pltpu.sync_copy(yt_hbm.at[pl.ds(tok0, CHUNK)], yidx[b])

                pltpu.async_copy(xtab_sp.at[xidx[b]], bufX[b], semB[b])
                pltpu.async_copy(ytab_sp.at[yidx[b]], bufY[b], semB[b])

                bj = (b - K) % NBUF
                cj = ci - K
                tokj = base_tok + cj * CHUNK

                rowj = wid * ROWS_PER_W + cj

                @pl.when(cj >= 0)
                def _(bj=bj, rowj=rowj):
                    combine_store(bj, rowj)

                @pl.when(jnp.logical_and(cj >= 0, cj + NBUF < NCHUNK))
                def _(bj=bj, tokj=tokj):
                    tok1 = tokj + NBUF * CHUNK
                    pltpu.async_copy(
                        xt_hbm.at[pl.ds(tok1, CHUNK)], xidx[bj], semA[bj])
                    pltpu.async_copy(
                        yt_hbm.at[pl.ds(tok1, CHUNK)], yidx[bj], semA[bj])
            return ()

        lax.fori_loop(0, NSUPER, superstep, ())

        for bj in range(NBUF - K, NBUF):
            cj = NCHUNK - NBUF + bj
            combine_store(bj, wid * ROWS_PER_W + cj)
        for b in range(NBUF):
            drain(semD[b], bufX[b])

    return embed_kernel


_kernel = _make_kernel()


@jax.jit
def kernel(x_tokens, y_tokens, x_table, y_table, pos_table):
    xt = x_tokens.reshape(N).astype(jnp.int32)
    yt = y_tokens.reshape(N).astype(jnp.int32)
    return _kernel(xt, yt, x_table, y_table, pos_table)
